# baseline (speedup 1.0000x reference)
import functools

import jax
import jax.numpy as jnp
from jax import lax
from jax.experimental import pallas as pl
from jax.experimental.pallas import tpu as pltpu
from jax.experimental.pallas import tpu_sc as plsc

VOCAB = 100000
CONTEXT = 20
EMBED = 32
HIDDEN = 128
BATCH = 1024

NIDX = BATCH * CONTEXT

BM = 32
NM = BATCH // BM
NSLOT = 2


@functools.cache
def _gather_call(n_idx, embed):
    info = plsc.get_sparse_core_info()
    nc, ns = info.num_cores, info.num_subcores
    nw = nc * ns
    assert n_idx % nw == 0
    b_per_w = n_idx // nw
    mesh = plsc.VectorSubcoreMesh(core_axis_name="c", subcore_axis_name="s")

    @functools.partial(
        pl.kernel,
        mesh=mesh,
        out_type=jax.ShapeDtypeStruct((n_idx, embed), jnp.float32),
        scratch_types=[
            pltpu.VMEM((b_per_w,), jnp.int32),
            pltpu.VMEM((b_per_w, embed), jnp.float32),
            pltpu.SemaphoreType.DMA,
        ],
        compiler_params=pltpu.CompilerParams(use_tc_tiling_on_sc=False),
    )
    def gather_k(idx_hbm, table_hbm, out_hbm, idx_v, rows_v, sem):
        wid = lax.axis_index("s") * nc + lax.axis_index("c")
        base = wid * b_per_w
        pltpu.sync_copy(idx_hbm.at[pl.ds(base, b_per_w)], idx_v)
        pltpu.async_copy(table_hbm.at[idx_v], rows_v, sem).wait()
        pltpu.sync_copy(rows_v, out_hbm.at[pl.ds(base, b_per_w)])

    return gather_k


def _hidden_body(emb_ref, w1_ref, b1_ref, hid_ref):
    h = jnp.dot(emb_ref[...], w1_ref[...], preferred_element_type=jnp.float32)
    hid_ref[...] = jnp.maximum(h + b1_ref[...], 0.0).astype(jnp.bfloat16)


def _logits_body(hid_ref, w2_ref, b2_ref, out_hbm, buf, sems):
    m = pl.program_id(0)
    slot = m % NSLOT

    @pl.when(m >= NSLOT)
    def _():
        pltpu.make_async_copy(
            buf.at[slot],
            out_hbm.at[pl.ds((m - NSLOT) * BM, BM)],
            sems.at[slot],
        ).wait()

    h = hid_ref[pl.ds(m * BM, BM), :]
    buf[slot] = jnp.dot(h, w2_ref[...],
                        preferred_element_type=jnp.float32) + b2_ref[...]

    pltpu.make_async_copy(
        buf.at[slot],
        out_hbm.at[pl.ds(m * BM, BM)],
        sems.at[slot],
    ).start()

    @pl.when(m == NM - 1)
    def _():
        for k in range(NSLOT):
            pltpu.make_async_copy(
                buf.at[(NM - 1 - k) % NSLOT],
                out_hbm.at[pl.ds((NM - 1 - k) * BM, BM)],
                sems.at[(NM - 1 - k) % NSLOT],
            ).wait()


def kernel(x, table, W1, b1, W2, b2):
    idx = x.reshape(-1).astype(jnp.int32)
    embedded = _gather_call(NIDX, EMBED)(idx, table)
    embedded = embedded.reshape(BATCH, CONTEXT * EMBED)

    hidden = pl.pallas_call(
        _hidden_body,
        out_shape=jax.ShapeDtypeStruct((BATCH, HIDDEN), jnp.bfloat16),
    )(embedded, W1, b1.reshape(1, HIDDEN))

    w2b = W2.astype(jnp.bfloat16)

    logits = pl.pallas_call(
        _logits_body,
        grid=(NM,),
        in_specs=[
            pl.BlockSpec((BATCH, HIDDEN), lambda m: (0, 0)),
            pl.BlockSpec((HIDDEN, VOCAB), lambda m: (0, 0)),
            pl.BlockSpec((1, VOCAB), lambda m: (0, 0)),
        ],
        out_specs=pl.BlockSpec(memory_space=pltpu.MemorySpace.HBM),
        out_shape=jax.ShapeDtypeStruct((BATCH, VOCAB), jnp.float32),
        scratch_shapes=[
            pltpu.VMEM((NSLOT, BM, VOCAB), jnp.float32),
            pltpu.SemaphoreType.DMA((NSLOT,)),
        ],
    )(hidden, w2b, b2.reshape(1, VOCAB))
    return logits


# speedup vs baseline: 1.0067x; 1.0067x over previous
import functools

import jax
import jax.numpy as jnp
from jax import lax
from jax.experimental import pallas as pl
from jax.experimental.pallas import tpu as pltpu
from jax.experimental.pallas import tpu_sc as plsc

VOCAB = 100000
CONTEXT = 20
EMBED = 32
HIDDEN = 128
BATCH = 1024

NIDX = BATCH * CONTEXT

BM = 32
NM = BATCH // BM
NSLOT = 2


@functools.cache
def _gather_call(n_idx, embed):
    info = plsc.get_sparse_core_info()
    nc, ns = info.num_cores, info.num_subcores
    nw = nc * ns
    assert n_idx % nw == 0
    b_per_w = n_idx // nw
    mesh = plsc.VectorSubcoreMesh(core_axis_name="c", subcore_axis_name="s")

    @functools.partial(
        pl.kernel,
        mesh=mesh,
        out_type=jax.ShapeDtypeStruct((n_idx, embed), jnp.float32),
        scratch_types=[
            pltpu.VMEM((b_per_w,), jnp.int32),
            pltpu.VMEM((b_per_w, embed), jnp.float32),
            pltpu.SemaphoreType.DMA,
        ],
        compiler_params=pltpu.CompilerParams(use_tc_tiling_on_sc=False),
    )
    def gather_k(idx_hbm, table_hbm, out_hbm, idx_v, rows_v, sem):
        wid = lax.axis_index("s") * nc + lax.axis_index("c")
        base = wid * b_per_w
        pltpu.sync_copy(idx_hbm.at[pl.ds(base, b_per_w)], idx_v)
        pltpu.async_copy(table_hbm.at[idx_v], rows_v, sem).wait()
        pltpu.sync_copy(rows_v, out_hbm.at[pl.ds(base, b_per_w)])

    return gather_k


def _hidden_body(emb_ref, w1_ref, b1_ref, hid_ref):
    h = jnp.dot(emb_ref[...], w1_ref[...], preferred_element_type=jnp.float32)
    hid_ref[...] = jnp.maximum(h + b1_ref[...], 0.0).astype(jnp.bfloat16)


HB = BM // 2


def _stripe_copies(out_hbm, buf, sems, slot, m):
    for h in range(2):
        yield pltpu.make_async_copy(
            buf.at[slot, pl.ds(h * HB, HB)],
            out_hbm.at[pl.ds(m * BM + h * HB, HB)],
            sems.at[slot],
        ), h


def _logits_body(hid_ref, w2_ref, b2_ref, out_hbm, buf, sems):
    m = pl.program_id(0)
    slot = m % NSLOT

    @pl.when(m >= NSLOT)
    def _():
        for c, _h in _stripe_copies(out_hbm, buf, sems, slot, m - NSLOT):
            c.wait()

    h = hid_ref[pl.ds(m * BM, BM), :]
    buf[slot] = jnp.dot(h, w2_ref[...],
                        preferred_element_type=jnp.float32) + b2_ref[...]

    for c, hh in _stripe_copies(out_hbm, buf, sems, slot, m):
        c.start(priority=hh)

    @pl.when(m == NM - 1)
    def _():
        for k in range(NSLOT):
            for c, _h in _stripe_copies(out_hbm, buf, sems,
                                        (NM - 1 - k) % NSLOT, NM - 1 - k):
                c.wait()


def kernel(x, table, W1, b1, W2, b2):
    idx = x.reshape(-1).astype(jnp.int32)
    embedded = _gather_call(NIDX, EMBED)(idx, table)
    embedded = embedded.reshape(BATCH, CONTEXT * EMBED)

    hidden = pl.pallas_call(
        _hidden_body,
        out_shape=jax.ShapeDtypeStruct((BATCH, HIDDEN), jnp.bfloat16),
    )(embedded, W1, b1.reshape(1, HIDDEN))

    w2b = W2.astype(jnp.bfloat16)

    logits = pl.pallas_call(
        _logits_body,
        grid=(NM,),
        in_specs=[
            pl.BlockSpec((BATCH, HIDDEN), lambda m: (0, 0)),
            pl.BlockSpec((HIDDEN, VOCAB), lambda m: (0, 0)),
            pl.BlockSpec((1, VOCAB), lambda m: (0, 0)),
        ],
        out_specs=pl.BlockSpec(memory_space=pltpu.MemorySpace.HBM),
        out_shape=jax.ShapeDtypeStruct((BATCH, VOCAB), jnp.float32),
        scratch_shapes=[
            pltpu.VMEM((NSLOT, BM, VOCAB), jnp.float32),
            pltpu.SemaphoreType.DMA((NSLOT,)),
        ],
    )(hidden, w2b, b2.reshape(1, VOCAB))
    return logits
